# baseline (speedup 1.0000x reference)
import functools

import jax
import jax.numpy as jnp
from jax import lax
from jax.experimental import pallas as pl
from jax.experimental.pallas import tpu as pltpu
from jax.experimental.pallas import tpu_sc as plsc

_NC = 2
_NS = 16
_NW = _NC * _NS

_BLK = 320
_NBUF = 4


def _make_gather(total_rows: int, embed_dim: int):
    per_w = total_rows // _NW
    n_blk = per_w // _BLK
    assert n_blk % _NBUF == 0
    mesh = plsc.VectorSubcoreMesh(core_axis_name="c", subcore_axis_name="s")

    @functools.partial(
        pl.kernel,
        mesh=mesh,
        out_type=jax.ShapeDtypeStruct((total_rows, embed_dim), jnp.float32),
        compiler_params=pltpu.CompilerParams(use_tc_tiling_on_sc=False),
        scratch_types=[
            pltpu.VMEM((per_w,), jnp.int32),
            pltpu.VMEM((_NBUF, _BLK, embed_dim), jnp.float32),
            [pltpu.SemaphoreType.DMA] * _NBUF,
            [pltpu.SemaphoreType.DMA] * _NBUF,
        ],
    )
    def gather_kernel(idx_hbm, table_hbm, out_hbm, idx_v, rows_v, gsem, ssem):
        wid = lax.axis_index("s") * _NC + lax.axis_index("c")
        base = wid * per_w

        pltpu.sync_copy(idx_hbm.at[pl.ds(base, per_w)], idx_v)

        def gather_copy(i, b):
            return pltpu.make_async_copy(
                table_hbm.at[idx_v.at[pl.ds(i * _BLK, _BLK)]],
                rows_v.at[b],
                gsem[b],
            )

        def store_copy(i, b):
            return pltpu.make_async_copy(
                rows_v.at[b],
                out_hbm.at[pl.ds(base + i * _BLK, _BLK)],
                ssem[b],
            )

        for b in range(_NBUF):
            gather_copy(b, b).start()

        def step(g, carry):
            for b in range(_NBUF):
                i = g * _NBUF + b
                gather_copy(i, b).wait()
                store_copy(i, b).start()
                store_copy(i, b).wait()

                @pl.when(i + _NBUF < n_blk)
                def _():
                    gather_copy(i + _NBUF, b).start()
            return carry

        lax.fori_loop(0, n_blk // _NBUF, step, 0)

    return gather_kernel


def kernel(lang, batch_input, table):
    del lang
    bsz, seq = batch_input.shape
    _, embed_dim = table.shape
    idx = batch_input.reshape(-1)
    out = _make_gather(bsz * seq, embed_dim)(idx, table)
    return out.reshape(bsz, seq, embed_dim)


# speedup vs baseline: 1.1457x; 1.1457x over previous
import functools

import jax
import jax.numpy as jnp
from jax import lax
from jax.experimental import pallas as pl
from jax.experimental.pallas import tpu as pltpu
from jax.experimental.pallas import tpu_sc as plsc

_NC = 2
_NS = 16
_NW = _NC * _NS

_LANES = 128
_NBUF = 2


def _make_gather(bsz: int, seq: int, embed_dim: int):
    dg = embed_dim // 8
    nbt = bsz // _LANES
    tiles_w = nbt // _NW
    rows_w = tiles_w * _LANES
    per_w = rows_w * seq
    n_blk = tiles_w * seq
    assert n_blk % _NBUF == 0
    mesh = plsc.VectorSubcoreMesh(core_axis_name="c", subcore_axis_name="s")

    @functools.partial(
        pl.kernel,
        mesh=mesh,
        out_type=jax.ShapeDtypeStruct((seq, dg, nbt, 8, _LANES), jnp.float32),
        compiler_params=pltpu.CompilerParams(
            use_tc_tiling_on_sc=False, needs_layout_passes=False
        ),
        scratch_types=[
            pltpu.VMEM((per_w,), jnp.int32),
            pltpu.VMEM((n_blk, _LANES), jnp.int32),
            pltpu.VMEM((_NBUF, _LANES, embed_dim), jnp.float32),
            pltpu.VMEM((_NBUF, embed_dim, _LANES + 1), jnp.float32),
            [pltpu.SemaphoreType.DMA] * _NBUF,
            [pltpu.SemaphoreType.DMA] * _NBUF,
        ],
    )
    def gather_kernel(idx_hbm, table_hbm, out_hbm, idx_v, blk_idx, gbuf, tbuf,
                      gsem, ssem):
        wid = lax.axis_index("s") * _NC + lax.axis_index("c")
        lanes16 = lax.iota(jnp.int32, 16)

        pltpu.sync_copy(idx_hbm.at[pl.ds(wid * per_w, per_w)], idx_v)

        def build(t, carry):
            bt = t // seq
            s_ = t - bt * seq
            base = bt * (_LANES * seq) + s_
            vals = [
                plsc.load_gather(idx_v, [base + (j0 * 16 + lanes16) * seq])
                for j0 in range(_LANES // 16)
            ]
            for j0 in range(_LANES // 16):
                blk_idx[t, pl.ds(j0 * 16, 16)] = vals[j0]
            return carry

        lax.fori_loop(0, n_blk, build, 0)

        def gather_copy(t, b):
            return pltpu.make_async_copy(
                table_hbm.at[blk_idx.at[t]], gbuf.at[b], gsem[b]
            )

        def store_copies(t, b):
            bt = t // seq
            s_ = t - bt * seq
            btile = wid * tiles_w + bt
            return [
                pltpu.make_async_copy(
                    tbuf.at[b].at[pl.ds(g * 8, 8), pl.ds(0, _LANES)],
                    out_hbm.at[s_, g, btile],
                    ssem[b],
                )
                for g in range(dg)
            ]

        dvecs = [d0 + lanes16 for d0 in range(0, embed_dim, 16)]

        nk = len(dvecs)

        def transpose(b):
            src = gbuf.at[b]
            dst = tbuf.at[b]

            def loads(j):
                return [src[j, pl.ds(k * 16, 16)] for k in range(nk)]

            def stores(j, vals):
                jvec = jnp.full((16,), j, jnp.int32)
                for k in range(nk):
                    plsc.store_scatter(dst, [dvecs[k], jvec], vals[k])

            prev = loads(0)
            for j in range(1, _LANES):
                cur = loads(j)
                stores(j - 1, prev)
                prev = cur
            stores(_LANES - 1, prev)

        for b in range(_NBUF):
            gather_copy(b, b).start()

        def step(u, carry):
            for b in range(_NBUF):
                t = u * _NBUF + b
                gather_copy(t, b).wait()

                @pl.when(t >= _NBUF)
                def _():
                    for c in store_copies(t - _NBUF, b):
                        c.wait()

                transpose(b)

                @pl.when(t + _NBUF < n_blk)
                def _():
                    gather_copy(t + _NBUF, b).start()

                for c in store_copies(t, b):
                    c.start()
            return carry

        lax.fori_loop(0, n_blk // _NBUF, step, 0)

        for b in range(_NBUF):
            for c in store_copies(n_blk - _NBUF + b, b):
                c.wait()

    return gather_kernel


def kernel(lang, batch_input, table):
    del lang
    bsz, seq = batch_input.shape
    _, embed_dim = table.shape
    idx = batch_input.reshape(-1)
    out5 = _make_gather(bsz, seq, embed_dim)(idx, table)
    out = out5.transpose(2, 4, 0, 1, 3).reshape(bsz, seq, embed_dim)
    return out
